# baseline (speedup 1.0000x reference)
import functools

import jax
import jax.numpy as jnp
from jax import lax
from jax.experimental import pallas as pl
from jax.experimental.pallas import tpu as pltpu
from jax.experimental.pallas import tpu_sc as plsc

_VOCAB = 100000
_EMB = 32
_B = 1024
_L = 200

_NC = 2
_NS = 16
_NW = _NC * _NS
_CH = 100
_LANES = 16
_NHALF = 1
_BH = _B // _NHALF
_BPW = _BH // _NW
_CPW = _BPW * _L // _CH

_mesh = plsc.VectorSubcoreMesh(core_axis_name="c", subcore_axis_name="s")


def _make_pool(half):
    base_chunk = half * (_BH * _L // _CH)

    @functools.partial(
        pl.kernel,
        mesh=_mesh,
        out_type=jax.ShapeDtypeStruct((_BH, _EMB), jnp.float32),
        scratch_types=(
            [pltpu.VMEM((_CPW, _CH), jnp.int32)]
            + [pltpu.VMEM((_L, _EMB), jnp.float32) for _ in range(2)]
            + [pltpu.VMEM((_BPW, _EMB), jnp.float32)]
            + [pltpu.SemaphoreType.DMA for _ in range(2)]
        ),
        compiler_params=pltpu.CompilerParams(use_tc_tiling_on_sc=False),
        name=f"pool_half{half}",
    )
    def _pool(x_hbm, table_hbm, out_hbm, idx_v, buf_a, buf_b, pooled_v,
              sem_a, sem_b):
        bufs = (buf_a, buf_b)
        sems = (sem_a, sem_b)
        wid = lax.axis_index("s") * _NC + lax.axis_index("c")
        pltpu.sync_copy(x_hbm.at[pl.ds(base_chunk + wid * _CPW, _CPW)], idx_v)
        inv_l = jnp.float32(1.0 / _L)

        def _fire_row(r, buf, sem):
            pltpu.async_copy(table_hbm.at[idx_v.at[2 * r]],
                             buf.at[pl.ds(0, _CH)], sem)
            pltpu.async_copy(table_hbm.at[idx_v.at[2 * r + 1]],
                             buf.at[pl.ds(_CH, _CH)], sem)

        def _wait_row(buf, sem):
            pltpu.make_async_copy(table_hbm.at[pl.ds(0, _L)], buf, sem).wait()

        def _accum(buf):
            def body(l, c):
                b = l * 8
                new = list(c)
                for u in range(8):
                    new[(2 * u) % 8] = (new[(2 * u) % 8] +
                                        buf[b + u, pl.ds(0, _LANES)])
                    new[(2 * u + 1) % 8] = (new[(2 * u + 1) % 8] +
                                            buf[b + u, pl.ds(_LANES, _LANES)])
                return tuple(new)
            z = jnp.zeros((_LANES,), jnp.float32)
            c = lax.fori_loop(0, _L // 8, body, (z,) * 8)
            return ((c[0] + c[2]) + (c[4] + c[6]),
                    (c[1] + c[3]) + (c[5] + c[7]))

        _fire_row(0, bufs[0], sems[0])
        _fire_row(1, bufs[1], sems[1])

        def pair_body(k, carry):
            for h in range(2):
                r = 2 * k + h
                buf, sem = bufs[h], sems[h]
                _wait_row(buf, sem)
                a_lo, a_hi = _accum(buf)

                @pl.when(k < _BPW // 2 - 1)
                def _prefetch():
                    _fire_row(r + 2, buf, sem)

                pooled_v[r, pl.ds(0, _LANES)] = a_lo * inv_l
                pooled_v[r, pl.ds(_LANES, _LANES)] = a_hi * inv_l
            return carry

        lax.fori_loop(0, _BPW // 2, pair_body, 0)
        pltpu.sync_copy(pooled_v, out_hbm.at[pl.ds(wid * _BPW, _BPW)])

    return _pool


_pools = [_make_pool(h) for h in range(_NHALF)]

_RB = 32
_NGRID = _B // _RB
_NBUF = 3


def _fc_body(p_ref, w_hbm, b_ref, o_hbm, w_v, ob0, ob1, ob2, semw,
             sem0, sem1, sem2):
    i = pl.program_id(0)
    obufs = (ob0, ob1, ob2)
    sems = (sem0, sem1, sem2)

    @pl.when(i == 0)
    def _load_w():
        cp = pltpu.make_async_copy(w_hbm, w_v, semw)
        cp.start()
        cp.wait()

    for s in range(_NBUF):
        @pl.when(i % _NBUF == s)
        def _slot():
            obuf, sem = obufs[s], sems[s]

            @pl.when(i >= _NBUF)
            def _drain_prev():
                pltpu.make_async_copy(
                    obuf, o_hbm.at[pl.ds(0, _RB)], sem).wait()

            obuf[...] = lax.dot_general(
                p_ref[...], w_v[...],
                dimension_numbers=(((1,), (0,)), ((), ())),
                preferred_element_type=jnp.float32,
            ) + b_ref[...]
            pltpu.make_async_copy(
                obuf, o_hbm.at[pl.ds(i * _RB, _RB)], sem).start()

            @pl.when(i == _NGRID - 1)
            def _final_drain():
                for t in range(_NBUF):
                    pltpu.make_async_copy(
                        obufs[(s + t) % _NBUF], o_hbm.at[pl.ds(0, _RB)],
                        sems[(s + t) % _NBUF]).wait()


def _fc(pooled, fc_w, fc_b2):
    return pl.pallas_call(
        _fc_body,
        grid=(_NGRID,),
        in_specs=[
            pl.BlockSpec((_RB, _EMB), lambda i: (i, 0)),
            pl.BlockSpec(memory_space=pl.ANY),
            pl.BlockSpec((1, _VOCAB), lambda i: (0, 0)),
        ],
        out_specs=pl.BlockSpec(memory_space=pl.ANY),
        out_shape=jax.ShapeDtypeStruct((_B, _VOCAB), jnp.float32),
        scratch_shapes=(
            [pltpu.VMEM((_EMB, _VOCAB), jnp.float32)]
            + [pltpu.VMEM((_RB, _VOCAB), jnp.float32)
               for _ in range(_NBUF)]
            + [pltpu.SemaphoreType.DMA for _ in range(_NBUF + 1)]
        ),
    )(pooled, fc_w, fc_b2)


def kernel(x, emb_table, fc_w, fc_b):
    xi = x.astype(jnp.int32).reshape(_B * _L // _CH, _CH)
    fc_b2 = fc_b.reshape(1, _VOCAB)
    pooled = _pools[0](xi, emb_table)
    return _fc(pooled, jnp.swapaxes(fc_w, 0, 1), fc_b2)
